# Initial kernel scaffold; baseline (speedup 1.0000x reference)
import jax
import jax.numpy as jnp
from jax.experimental import pallas as pl


def kernel(x, token_embedding, cls, position_embedding):
    raise NotImplementedError("write your pallas kernel here")


# speedup vs baseline: 7.1971x; 7.1971x over previous
import jax
import jax.numpy as jnp
from jax.experimental import pallas as pl


def _tc_body(x_ref, tok_ref, cls_ref, pos_ref, o_ref):
    BB, T = x_ref.shape
    D = tok_ref.shape[1]
    xf = x_ref[...].astype(jnp.float32)
    e0 = tok_ref[0, :]
    diff = tok_ref[1, :] - e0
    base = pos_ref[0, 0:T, :] + e0[None, :]
    o_ref[:, 0:T, :] = base[None] + xf[:, :, None] * diff[None, None, :]
    cls_row = cls_ref[0, 0, :] + pos_ref[0, T, :]
    o_ref[:, T, :] = jnp.broadcast_to(cls_row[None, :], (BB, D))


def kernel(x, token_embedding, cls, position_embedding):
    B, T = x.shape
    D = token_embedding.shape[1]
    BB = 256
    out = pl.pallas_call(
        _tc_body,
        grid=(B // BB,),
        in_specs=[
            pl.BlockSpec((BB, T), lambda i: (i, 0)),
            pl.BlockSpec((2, D), lambda i: (0, 0)),
            pl.BlockSpec((1, 1, D), lambda i: (0, 0, 0)),
            pl.BlockSpec((1, T + 1, D), lambda i: (0, 0, 0)),
        ],
        out_specs=pl.BlockSpec((BB, T + 1, D), lambda i: (i, 0, 0)),
        out_shape=jax.ShapeDtypeStruct((B, T + 1, D), jnp.float32),
    )(x.astype(jnp.int32), token_embedding, cls, position_embedding)
    return out


# speedup vs baseline: 10.6983x; 1.4865x over previous
import functools

import jax
import jax.numpy as jnp
from jax import lax
from jax.experimental import pallas as pl
from jax.experimental.pallas import tpu as pltpu
from jax.experimental.pallas import tpu_sc as plsc

_NC = 2
_NS = 16
_L = 16

_B = 4096
_T = 200
_D = 64
_ROW = (_T + 1) * _D
_NW = _NC * _NS
_RPW = _B // _NW
_XPW = _RPW * _T
_NBUF = 2


def _sc_body(x_hbm, tok_hbm, cls_hbm, pos_hbm, out_hbm,
             posv, xv, outb0, outb1, sem0, sem1):
    wid = lax.axis_index("s") * _NC + lax.axis_index("c")
    rb = wid * _RPW
    outb = (outb0, outb1)
    sems = (sem0, sem1)

    pltpu.sync_copy(pos_hbm, posv.at[pl.ds(0, _ROW)])
    pltpu.sync_copy(tok_hbm, posv.at[pl.ds(_ROW, 2 * _D)])
    pltpu.sync_copy(cls_hbm, posv.at[pl.ds(_ROW + 2 * _D, _D)])
    pltpu.sync_copy(x_hbm.at[pl.ds(wid * _XPW, _XPW)], xv.at[pl.ds(0, _XPW)])

    e0 = [posv[pl.ds(_ROW + c * _L, _L)] for c in range(4)]
    e1 = [posv[pl.ds(_ROW + _D + c * _L, _L)] for c in range(4)]
    clsv = [posv[pl.ds(_ROW + 2 * _D + c * _L, _L)] for c in range(4)]
    diffs = [e1[c] - e0[c] for c in range(4)]

    def fold(t, carry):
        for c in range(4):
            o = t * _D + c * _L
            posv[pl.ds(o, _L)] = posv[pl.ds(o, _L)] + e0[c]
        return carry
    lax.fori_loop(0, _T, fold, 0)
    for c in range(4):
        o = _T * _D + c * _L
        posv[pl.ds(o, _L)] = posv[pl.ds(o, _L)] + clsv[c]

    def emit_token(ob, xf, j, tb):
        xb = lax.gather(
            xf, jnp.full((_L, 1), j, jnp.int32),
            lax.GatherDimensionNumbers(offset_dims=(), collapsed_slice_dims=(0,),
                                       start_index_map=(0,)),
            (1,), mode=lax.GatherScatterMode.PROMISE_IN_BOUNDS)
        for c in range(4):
            o = tb + c * _L
            ob[pl.ds(o, _L)] = posv[pl.ds(o, _L)] + xb * diffs[c]

    def compute_row(lb, ob):
        xrow = lb * _T

        def chunk(tc, carry):
            xf = xv[pl.ds(xrow + tc * _L, _L)].astype(jnp.float32)
            for j in range(_L):
                emit_token(ob, xf, j, (tc * _L + j) * _D)
            return carry
        lax.fori_loop(0, _T // _L, chunk, 0)
        nfull = (_T // _L) * _L
        xf = xv[pl.ds(xrow + nfull, _L)].astype(jnp.float32)
        for j in range(_T - nfull):
            emit_token(ob, xf, j, (nfull + j) * _D)
        for c in range(4):
            o = _T * _D + c * _L
            ob[pl.ds(o, _L)] = posv[pl.ds(o, _L)]

    def group(g, carry):
        for k in range(_NBUF):
            lb = g * _NBUF + k

            @pl.when(g > 0)
            def _wait():
                pltpu.make_async_copy(outb[k], out_hbm.at[rb + lb],
                                      sems[k]).wait()

            compute_row(lb, outb[k])
            pltpu.async_copy(outb[k], out_hbm.at[rb + lb], sems[k])
        return carry

    lax.fori_loop(0, _RPW // _NBUF, group, 0)
    for k in range(_NBUF):
        pltpu.make_async_copy(outb[k], out_hbm.at[rb], sems[k]).wait()


def kernel(x, token_embedding, cls, position_embedding):
    B, T = x.shape
    D = token_embedding.shape[1]
    mesh = plsc.VectorSubcoreMesh(core_axis_name="c", subcore_axis_name="s")
    run = pl.kernel(
        _sc_body,
        mesh=mesh,
        out_type=jax.ShapeDtypeStruct((B, _ROW), jnp.float32),
        scratch_types=[
            pltpu.VMEM((_ROW + 3 * _D,), jnp.float32),
            pltpu.VMEM((_XPW + _L,), jnp.int32),
            pltpu.VMEM((_ROW,), jnp.float32),
            pltpu.VMEM((_ROW,), jnp.float32),
            pltpu.SemaphoreType.DMA,
            pltpu.SemaphoreType.DMA,
        ],
    )
    out = run(
        x.astype(jnp.int32).reshape(-1),
        token_embedding.reshape(-1),
        cls.reshape(-1),
        position_embedding.reshape(-1),
    )
    return out.reshape(B, T + 1, D)
